# Initial kernel scaffold; baseline (speedup 1.0000x reference)
import jax
import jax.numpy as jnp
from jax.experimental import pallas as pl


def kernel(h, embeddings, gumbels):
    raise NotImplementedError("write your pallas kernel here")


# speedup vs baseline: 2.9830x; 2.9830x over previous
import functools

import jax
import jax.numpy as jnp
from jax.experimental import pallas as pl
import jax.experimental.pallas.tpu as pltpu


def _fused_body(h_ref, emb_ref, gum_ref,
                soft_ref, hard_ref, logits_ref, ids_ref, ent_ref,
                vn_ref):
    @pl.when(pl.program_id(0) == 0)
    def _normalize_codebook():
        v = emb_ref[...]
        n = jnp.sqrt(jnp.sum(v * v, axis=1, keepdims=True))
        vn_ref[...] = v / jnp.maximum(n, 1e-12)

    h = h_ref[...]
    hn = h / jnp.maximum(
        jnp.sqrt(jnp.sum(h * h, axis=1, keepdims=True)), 1e-12)
    logits = jax.lax.dot_general(
        hn, vn_ref[...],
        dimension_numbers=(((1,), (1,)), ((), ())),
        preferred_element_type=jnp.float32)
    logits_ref[...] = logits

    z = logits + gum_ref[...]
    m2 = jnp.max(z, axis=-1, keepdims=True)
    e2 = jnp.exp(z - m2)
    soft = e2 / jnp.sum(e2, axis=-1, keepdims=True)
    soft_ref[...] = soft

    idx2 = jnp.argmax(soft, axis=-1)
    k_iota = jax.lax.broadcasted_iota(jnp.int32, soft.shape, 1)
    hard_ref[...] = (k_iota == idx2[:, None]).astype(jnp.float32)

    m1 = jnp.max(logits, axis=-1, keepdims=True)
    e1 = jnp.exp(logits - m1)
    p = e1 / jnp.sum(e1, axis=-1, keepdims=True)
    ent_ref[...] = -jnp.sum(p * jnp.log(p + 1e-10), axis=-1)
    ids_ref[...] = jnp.argmax(logits, axis=-1)


@functools.partial(jax.jit, static_argnames=("tile_m",))
def _fused(h2, embeddings, gum2, tile_m):
    M, D = h2.shape
    K = embeddings.shape[0]
    grid = (M // tile_m,)
    out = pl.pallas_call(
        _fused_body,
        grid=grid,
        in_specs=[
            pl.BlockSpec((tile_m, D), lambda i: (i, 0)),
            pl.BlockSpec((K, D), lambda i: (0, 0)),
            pl.BlockSpec((tile_m, K), lambda i: (i, 0)),
        ],
        out_specs=[
            pl.BlockSpec((tile_m, K), lambda i: (i, 0)),
            pl.BlockSpec((tile_m, K), lambda i: (i, 0)),
            pl.BlockSpec((tile_m, K), lambda i: (i, 0)),
            pl.BlockSpec((tile_m,), lambda i: (i,)),
            pl.BlockSpec((tile_m,), lambda i: (i,)),
        ],
        out_shape=[
            jax.ShapeDtypeStruct((M, K), jnp.float32),
            jax.ShapeDtypeStruct((M, K), jnp.float32),
            jax.ShapeDtypeStruct((M, K), jnp.float32),
            jax.ShapeDtypeStruct((M,), jnp.int32),
            jax.ShapeDtypeStruct((M,), jnp.float32),
        ],
        scratch_shapes=[pltpu.VMEM((K, D), jnp.float32)],
        compiler_params=pltpu.CompilerParams(
            dimension_semantics=("arbitrary",),
        ),
    )(h2, embeddings, gum2)
    return out


def kernel(h, embeddings, gumbels):
    B, S, D = h.shape
    K = embeddings.shape[0]
    M = B * S
    tile_m = 256 if M % 256 == 0 else M
    soft, hard, logits, ids, ent = _fused(
        h.reshape(M, D), embeddings, gumbels.reshape(M, K), tile_m)
    return (soft.reshape(B, S, K), hard.reshape(B, S, K), ids.reshape(B, S),
            logits.reshape(B, S, K), ent.reshape(B, S))


# speedup vs baseline: 3.2457x; 1.0881x over previous
import functools

import jax
import jax.numpy as jnp
from jax.experimental import pallas as pl
import jax.experimental.pallas.tpu as pltpu


def _fused_body(h_ref, emb_ref, gum_ref,
                soft_ref, hard_ref, logits_ref, ids_ref, ent_ref,
                vn_ref):
    @pl.when(pl.program_id(0) == 0)
    def _normalize_codebook():
        v = emb_ref[...]
        n = jnp.sqrt(jnp.sum(v * v, axis=1, keepdims=True))
        vn_ref[...] = v / jnp.maximum(n, 1e-12)

    h = h_ref[...]
    inv_n = 1.0 / jnp.maximum(
        jnp.sqrt(jnp.sum(h * h, axis=1, keepdims=True)), 1e-12)
    hn = h * inv_n
    logits = jax.lax.dot_general(
        hn, vn_ref[...],
        dimension_numbers=(((1,), (1,)), ((), ())),
        preferred_element_type=jnp.float32)
    logits_ref[...] = logits

    z = logits + gum_ref[...]
    m2 = jnp.max(z, axis=-1, keepdims=True)
    e2 = jnp.exp(z - m2)
    s2 = jnp.sum(e2, axis=-1, keepdims=True)
    soft = e2 * (1.0 / s2)
    soft_ref[...] = soft

    hard_ref[...] = (z == m2).astype(jnp.float32)

    m1 = jnp.max(logits, axis=-1, keepdims=True)
    lc = logits - m1
    e1 = jnp.exp(lc)
    s1 = jnp.sum(e1, axis=-1, keepdims=True)
    ent = jnp.log(s1) - jnp.sum(e1 * lc, axis=-1, keepdims=True) / s1
    ent_ref[...] = ent[:, 0]
    ids_ref[...] = jnp.argmax(logits, axis=-1)


@functools.partial(jax.jit, static_argnames=("tile_m",))
def _fused(h2, embeddings, gum2, tile_m):
    M, D = h2.shape
    K = embeddings.shape[0]
    grid = (M // tile_m,)
    out = pl.pallas_call(
        _fused_body,
        grid=grid,
        in_specs=[
            pl.BlockSpec((tile_m, D), lambda i: (i, 0)),
            pl.BlockSpec((K, D), lambda i: (0, 0)),
            pl.BlockSpec((tile_m, K), lambda i: (i, 0)),
        ],
        out_specs=[
            pl.BlockSpec((tile_m, K), lambda i: (i, 0)),
            pl.BlockSpec((tile_m, K), lambda i: (i, 0)),
            pl.BlockSpec((tile_m, K), lambda i: (i, 0)),
            pl.BlockSpec((tile_m,), lambda i: (i,)),
            pl.BlockSpec((tile_m,), lambda i: (i,)),
        ],
        out_shape=[
            jax.ShapeDtypeStruct((M, K), jnp.float32),
            jax.ShapeDtypeStruct((M, K), jnp.float32),
            jax.ShapeDtypeStruct((M, K), jnp.float32),
            jax.ShapeDtypeStruct((M,), jnp.int32),
            jax.ShapeDtypeStruct((M,), jnp.float32),
        ],
        scratch_shapes=[pltpu.VMEM((K, D), jnp.float32)],
        compiler_params=pltpu.CompilerParams(
            dimension_semantics=("arbitrary",),
        ),
    )(h2, embeddings, gum2)
    return out


def kernel(h, embeddings, gumbels):
    B, S, D = h.shape
    K = embeddings.shape[0]
    M = B * S
    tile_m = 256 if M % 256 == 0 else M
    soft, hard, logits, ids, ent = _fused(
        h.reshape(M, D), embeddings, gumbels.reshape(M, K), tile_m)
    return (soft.reshape(B, S, K), hard.reshape(B, S, K), ids.reshape(B, S),
            logits.reshape(B, S, K), ent.reshape(B, S))


# speedup vs baseline: 3.3601x; 1.0353x over previous
import functools

import jax
import jax.numpy as jnp
from jax.experimental import pallas as pl
import jax.experimental.pallas.tpu as pltpu


def _pair_body(h_ref, emb_ref, gum_ref,
               soft_ref, hard_ref, logits_ref, ids_ref, ent_ref,
               vn_ref, sa_ref, sb_ref):
    @pl.when(pl.program_id(0) == 0)
    def _normalize_codebook():
        v = emb_ref[...]
        n = jnp.sqrt(jnp.sum(v * v, axis=1, keepdims=True))
        vn_ref[...] = v / jnp.maximum(n, 1e-12)

    tm = sa_ref.shape[0]

    def _epilogue(lg, half):
        rows = pl.ds(half * tm, tm)
        logits_ref[rows, :] = lg
        z = lg + gum_ref[rows, :]
        m2 = jnp.max(z, axis=-1, keepdims=True)
        e2 = jnp.exp(z - m2)
        s2 = jnp.sum(e2, axis=-1, keepdims=True)
        soft_ref[rows, :] = e2 * (1.0 / s2)
        hard_ref[rows, :] = (z == m2).astype(jnp.float32)

        m1 = jnp.max(lg, axis=-1, keepdims=True)
        lc = lg - m1
        e1 = jnp.exp(lc)
        s1 = jnp.sum(e1, axis=-1, keepdims=True)
        ent = jnp.log(s1) - jnp.sum(e1 * lc, axis=-1, keepdims=True) / s1
        ent_ref[rows] = ent[:, 0]
        ids_ref[rows] = jnp.argmax(lg, axis=-1)

    _epilogue(sa_ref[...], 0)
    _epilogue(sb_ref[...], 1)

    def _project(half, out_ref):
        h = h_ref[pl.ds(half * tm, tm), :]
        inv_n = 1.0 / jnp.maximum(
            jnp.sqrt(jnp.sum(h * h, axis=1, keepdims=True)), 1e-12)
        out_ref[...] = jax.lax.dot_general(
            h * inv_n, vn_ref[...],
            dimension_numbers=(((1,), (1,)), ((), ())),
            preferred_element_type=jnp.float32)

    _project(0, sa_ref)
    _project(1, sb_ref)


@functools.partial(jax.jit, static_argnames=("tile_m",))
def _fused(h2, embeddings, gum2, tile_m):
    M, D = h2.shape
    K = embeddings.shape[0]
    pair = 2 * tile_m
    n_pairs = M // pair
    grid = (n_pairs + 1,)
    last = n_pairs - 1

    def in_idx(i):
        return (jnp.minimum(i, last), 0)

    def out_idx(i):
        return (jnp.maximum(i - 1, 0), 0)

    def out_idx1(i):
        return (jnp.maximum(i - 1, 0),)

    out = pl.pallas_call(
        _pair_body,
        grid=grid,
        in_specs=[
            pl.BlockSpec((pair, D), in_idx),
            pl.BlockSpec((K, D), lambda i: (0, 0)),
            pl.BlockSpec((pair, K), out_idx),
        ],
        out_specs=[
            pl.BlockSpec((pair, K), out_idx),
            pl.BlockSpec((pair, K), out_idx),
            pl.BlockSpec((pair, K), out_idx),
            pl.BlockSpec((pair,), out_idx1),
            pl.BlockSpec((pair,), out_idx1),
        ],
        out_shape=[
            jax.ShapeDtypeStruct((M, K), jnp.float32),
            jax.ShapeDtypeStruct((M, K), jnp.float32),
            jax.ShapeDtypeStruct((M, K), jnp.float32),
            jax.ShapeDtypeStruct((M,), jnp.int32),
            jax.ShapeDtypeStruct((M,), jnp.float32),
        ],
        scratch_shapes=[
            pltpu.VMEM((K, D), jnp.float32),
            pltpu.VMEM((tile_m, K), jnp.float32),
            pltpu.VMEM((tile_m, K), jnp.float32),
        ],
        compiler_params=pltpu.CompilerParams(
            dimension_semantics=("arbitrary",),
        ),
    )(h2, embeddings, gum2)
    return out


def kernel(h, embeddings, gumbels):
    B, S, D = h.shape
    K = embeddings.shape[0]
    M = B * S
    tile_m = 256 if M % 512 == 0 else M // 2
    soft, hard, logits, ids, ent = _fused(
        h.reshape(M, D), embeddings, gumbels.reshape(M, K), tile_m)
    return (soft.reshape(B, S, K), hard.reshape(B, S, K), ids.reshape(B, S),
            logits.reshape(B, S, K), ent.reshape(B, S))


# speedup vs baseline: 3.4002x; 1.0119x over previous
import functools

import jax
import jax.numpy as jnp
from jax.experimental import pallas as pl
import jax.experimental.pallas.tpu as pltpu


def _pair_body(h_ref, emb_ref, gum_ref,
               soft_ref, hard_ref, logits_ref, ids_ref, ent_ref,
               vn_ref, sa_ref, sb_ref):
    @pl.when(pl.program_id(0) == 0)
    def _normalize_codebook():
        v = emb_ref[...]
        inv = 1.0 / jnp.maximum(
            jnp.sqrt(jnp.sum(v * v, axis=1, keepdims=True)), 1e-12)
        vn_ref[...] = v * inv

    tm = sa_ref.shape[0]

    def _epilogue(lg, half):
        rows = pl.ds(half * tm, tm)
        logits_ref[rows, :] = lg
        z = lg + gum_ref[rows, :]
        m2 = jnp.max(z, axis=-1, keepdims=True)
        e2 = jnp.exp(z)
        s2 = jnp.sum(e2, axis=-1, keepdims=True)
        soft_ref[rows, :] = e2 * (1.0 / s2)
        hard_ref[rows, :] = (z == m2).astype(jnp.float32)

        e1 = jnp.exp(lg)
        s1 = jnp.sum(e1, axis=-1, keepdims=True)
        ent = jnp.log(s1) - jnp.sum(e1 * lg, axis=-1, keepdims=True) / s1
        ent_ref[rows] = ent[:, 0]
        ids_ref[rows] = jnp.argmax(lg, axis=-1)

    _epilogue(sa_ref[...], 0)
    _epilogue(sb_ref[...], 1)

    def _project(half, out_ref):
        h = h_ref[pl.ds(half * tm, tm), :]
        inv_n = 1.0 / jnp.maximum(
            jnp.sqrt(jnp.sum(h * h, axis=1, keepdims=True)), 1e-12)
        out_ref[...] = jax.lax.dot_general(
            h * inv_n, vn_ref[...],
            dimension_numbers=(((1,), (1,)), ((), ())),
            preferred_element_type=jnp.float32)

    _project(0, sa_ref)
    _project(1, sb_ref)


@functools.partial(jax.jit, static_argnames=("tile_m",))
def _fused(h2, embeddings, gum2, tile_m):
    M, D = h2.shape
    K = embeddings.shape[0]
    pair = 2 * tile_m
    n_pairs = M // pair
    grid = (n_pairs + 1,)
    last = n_pairs - 1

    def in_idx(i):
        return (jnp.minimum(i, last), 0)

    def out_idx(i):
        return (jnp.maximum(i - 1, 0), 0)

    def out_idx1(i):
        return (jnp.maximum(i - 1, 0),)

    out = pl.pallas_call(
        _pair_body,
        grid=grid,
        in_specs=[
            pl.BlockSpec((pair, D), in_idx),
            pl.BlockSpec((K, D), lambda i: (0, 0)),
            pl.BlockSpec((pair, K), out_idx),
        ],
        out_specs=[
            pl.BlockSpec((pair, K), out_idx),
            pl.BlockSpec((pair, K), out_idx),
            pl.BlockSpec((pair, K), out_idx),
            pl.BlockSpec((pair,), out_idx1),
            pl.BlockSpec((pair,), out_idx1),
        ],
        out_shape=[
            jax.ShapeDtypeStruct((M, K), jnp.float32),
            jax.ShapeDtypeStruct((M, K), jnp.float32),
            jax.ShapeDtypeStruct((M, K), jnp.float32),
            jax.ShapeDtypeStruct((M,), jnp.int32),
            jax.ShapeDtypeStruct((M,), jnp.float32),
        ],
        scratch_shapes=[
            pltpu.VMEM((K, D), jnp.float32),
            pltpu.VMEM((tile_m, K), jnp.float32),
            pltpu.VMEM((tile_m, K), jnp.float32),
        ],
        compiler_params=pltpu.CompilerParams(
            dimension_semantics=("arbitrary",),
        ),
    )(h2, embeddings, gum2)
    return out


def kernel(h, embeddings, gumbels):
    B, S, D = h.shape
    K = embeddings.shape[0]
    M = B * S
    tile_m = 256 if M % 512 == 0 else M // 2
    soft, hard, logits, ids, ent = _fused(
        h.reshape(M, D), embeddings, gumbels.reshape(M, K), tile_m)
    return (soft.reshape(B, S, K), hard.reshape(B, S, K), ids.reshape(B, S),
            logits.reshape(B, S, K), ent.reshape(B, S))


# speedup vs baseline: 3.6922x; 1.0859x over previous
import functools

import jax
import jax.numpy as jnp
from jax.experimental import pallas as pl
import jax.experimental.pallas.tpu as pltpu


def _body(h_ref, emb_ref, gum_ref,
          soft_ref, hard_ref, logits_ref, ids_ref, ent_ref,
          vn_ref, s_ref):
    @pl.when(pl.program_id(0) == 0)
    def _normalize_codebook():
        v = emb_ref[...]
        inv = 1.0 / jnp.maximum(
            jnp.sqrt(jnp.sum(v * v, axis=1, keepdims=True)), 1e-12)
        vn_ref[...] = v * inv

    tm = h_ref.shape[0]
    i = pl.program_id(0)
    slot = jax.lax.rem(i, 2)

    lg = s_ref[pl.ds((1 - slot) * tm, tm), :]
    logits_ref[...] = lg
    z = lg + gum_ref[...]
    m2 = jnp.max(z, axis=-1, keepdims=True)
    e2 = jnp.exp(z)
    s2 = jnp.sum(e2, axis=-1, keepdims=True)
    soft_ref[...] = e2 * (1.0 / s2)
    hard_ref[...] = (z == m2).astype(jnp.float32)

    e1 = jnp.exp(lg)
    s1 = jnp.sum(e1, axis=-1, keepdims=True)
    ent = jnp.log(s1) - jnp.sum(e1 * lg, axis=-1, keepdims=True) / s1
    ent_ref[...] = ent[:, 0]
    ids_ref[...] = jnp.argmax(lg, axis=-1)

    h = h_ref[...]
    inv_n = 1.0 / jnp.maximum(
        jnp.sqrt(jnp.sum(h * h, axis=1, keepdims=True)), 1e-12)
    s_ref[pl.ds(slot * tm, tm), :] = jax.lax.dot_general(
        h * inv_n, vn_ref[...],
        dimension_numbers=(((1,), (1,)), ((), ())),
        preferred_element_type=jnp.float32)


@functools.partial(jax.jit, static_argnames=("tile_m",))
def _fused(h2, embeddings, gum2, tile_m):
    M, D = h2.shape
    K = embeddings.shape[0]
    n_tiles = M // tile_m
    grid = (n_tiles + 1,)
    last = n_tiles - 1

    def in_idx(i):
        return (jnp.minimum(i, last), 0)

    def out_idx(i):
        return (jnp.maximum(i - 1, 0), 0)

    def out_idx1(i):
        return (jnp.maximum(i - 1, 0),)

    out = pl.pallas_call(
        _body,
        grid=grid,
        in_specs=[
            pl.BlockSpec((tile_m, D), in_idx),
            pl.BlockSpec((K, D), lambda i: (0, 0)),
            pl.BlockSpec((tile_m, K), out_idx),
        ],
        out_specs=[
            pl.BlockSpec((tile_m, K), out_idx),
            pl.BlockSpec((tile_m, K), out_idx),
            pl.BlockSpec((tile_m, K), out_idx),
            pl.BlockSpec((tile_m,), out_idx1),
            pl.BlockSpec((tile_m,), out_idx1),
        ],
        out_shape=[
            jax.ShapeDtypeStruct((M, K), jnp.float32),
            jax.ShapeDtypeStruct((M, K), jnp.float32),
            jax.ShapeDtypeStruct((M, K), jnp.float32),
            jax.ShapeDtypeStruct((M,), jnp.int32),
            jax.ShapeDtypeStruct((M,), jnp.float32),
        ],
        scratch_shapes=[
            pltpu.VMEM((K, D), jnp.float32),
            pltpu.VMEM((2 * tile_m, K), jnp.float32),
        ],
        compiler_params=pltpu.CompilerParams(
            dimension_semantics=("arbitrary",),
        ),
    )(h2, embeddings, gum2)
    return out


def kernel(h, embeddings, gumbels):
    B, S, D = h.shape
    K = embeddings.shape[0]
    M = B * S
    tile_m = 512 if M % 512 == 0 else M
    soft, hard, logits, ids, ent = _fused(
        h.reshape(M, D), embeddings, gumbels.reshape(M, K), tile_m)
    return (soft.reshape(B, S, K), hard.reshape(B, S, K), ids.reshape(B, S),
            logits.reshape(B, S, K), ent.reshape(B, S))
